# baseline (speedup 1.0000x reference)
import functools

import jax
import jax.numpy as jnp
from jax import lax
from jax.experimental import pallas as pl
from jax.experimental.pallas import tpu as pltpu
from jax.experimental.pallas import tpu_sc as plsc

_L16 = 16
_NC = 2
_NS = 16
_NW = _NC * _NS
_REP = 8


def _make_sc_kernel(batch, ld):
    rows_per_w = batch // _NW
    ndma = rows_per_w // _REP
    nvec = ld // _L16
    mesh = plsc.VectorSubcoreMesh(core_axis_name="c", subcore_axis_name="s")

    @functools.partial(
        pl.kernel,
        mesh=mesh,
        out_type=jax.ShapeDtypeStruct((batch, ld), jnp.float32),
        scratch_types=[
            pltpu.VMEM((1, ld), jnp.float32),
            pltpu.VMEM((1, ld), jnp.float32),
            pltpu.VMEM((_REP, ld), jnp.float32),
            pltpu.SemaphoreType.DMA,
        ],
    )
    def sc_kernel(emb_hbm, pos_hbm, out_hbm, emb_v, pos_v, rep_v, sem):
        wid = lax.axis_index("s") * _NC + lax.axis_index("c")
        base = wid * rows_per_w
        pltpu.sync_copy(emb_hbm, emb_v)
        pltpu.sync_copy(pos_hbm, pos_v)

        def add_body(i, _):
            sl = pl.ds(i * _L16, _L16)
            v = emb_v[0, sl] + pos_v[0, sl]
            for r in range(_REP):
                rep_v[r, sl] = v
            return 0

        lax.fori_loop(0, nvec, add_body, 0)

        copies = [
            pltpu.make_async_copy(
                rep_v, out_hbm.at[pl.ds(base + j * _REP, _REP)], sem
            )
            for j in range(ndma)
        ]
        for c in copies:
            c.start()
        for c in copies:
            c.wait()

    return sc_kernel


def kernel(input_char, emb_table, pos_table):
    batch, length = input_char.shape
    d = emb_table.shape[1]
    ld = length * d
    emb_flat = emb_table[:length].reshape(1, ld)
    pos_flat = pos_table.reshape(1, -1)[:, :ld]
    sc_kernel = _make_sc_kernel(batch, ld)
    out = sc_kernel(emb_flat, pos_flat)
    return out.reshape(batch, length, d)
